# baseline (speedup 1.0000x reference)
import functools

import jax
import jax.numpy as jnp
from jax import lax
from jax.experimental import pallas as pl
from jax.experimental.pallas import tpu as pltpu
from jax.experimental.pallas import tpu_sc as plsc

N = 10000
E = 320000
D = 128
H = 128
K = 30
KP = 32
REG = 0.01

CH = 128
NCH = E // CH
NC = 2
NS = 16
NW = NC * NS
CH_BASE = NCH // NW
CH_EXTRA = NCH - CH_BASE * NW
SLAB = CH_BASE + 1
NP = 10240
RPT = NP // NS

_mesh = plsc.VectorSubcoreMesh(core_axis_name="c", subcore_axis_name="s")
_sc_params = pltpu.CompilerParams(needs_layout_passes=False,
                                  use_tc_tiling_on_sc=False)


def _make_spmm(width, table_in_spmem):
    tw = width // 16
    scratch = [
        pltpu.VMEM((SLAB * CH,), jnp.int32),
        pltpu.VMEM((CH,), jnp.int32),
        pltpu.VMEM((CH,), jnp.int32),
        pltpu.VMEM((CH,), jnp.float32),
        pltpu.VMEM((CH,), jnp.float32),
        pltpu.VMEM((CH, width), jnp.float32),
        pltpu.VMEM((CH, width), jnp.float32),
        pltpu.VMEM_SHARED((NP, width), jnp.float32),
        pltpu.SemaphoreType.DMA,
        pltpu.SemaphoreType.DMA,
        pltpu.SemaphoreType.DMA,
        pltpu.SemaphoreType.DMA,
    ]
    if table_in_spmem:
        scratch.append(pltpu.VMEM_SHARED((NP, width), jnp.float32))

    @functools.partial(
        pl.kernel,
        out_type=jax.ShapeDtypeStruct((2 * NP, width), jnp.float32),
        mesh=_mesh,
        compiler_params=_sc_params,
        scratch_types=scratch,
    )
    def spmm(y_hbm, col_hbm, row_hbm, av_hbm, out_hbm,
             colv, rowca, rowcb, avca, avcb, bufa, bufb, acc,
             gsema, gsemb, isema, isemb, *rest):
        table = rest[0] if table_in_spmem else y_hbm
        cid = lax.axis_index("c")
        sid = lax.axis_index("s")
        wid = sid * NC + cid
        start = wid * CH_BASE + jnp.minimum(wid, CH_EXTRA)
        count = CH_BASE + (wid < CH_EXTRA).astype(jnp.int32)

        pltpu.sync_copy(col_hbm.at[pl.ds(start * CH, SLAB * CH)], colv)

        zeros16 = jnp.zeros((16,), jnp.float32)

        def zrow(i, carry):
            for t in range(tw):
                bufa[i, pl.ds(t * 16, 16)] = zeros16
            return carry

        lax.fori_loop(0, CH, zrow, 0)
        base_row = sid * RPT
        for t in range(RPT // CH):
            pltpu.sync_copy(bufa, acc.at[pl.ds(base_row + t * CH, CH)])

        def gstart(j, buf, gsem):
            pltpu.async_copy(table.at[colv.at[pl.ds(j * CH, CH)]], buf, gsem)

        def gwait(buf, gsem):
            pltpu.make_async_copy(
                table.at[colv.at[pl.ds(0, CH)]], buf, gsem).wait()

        def istart(j, rowc, avc, isem):
            pltpu.async_copy(row_hbm.at[pl.ds((start + j) * CH, CH)],
                             rowc, isem)
            pltpu.async_copy(av_hbm.at[pl.ds((start + j) * CH, CH)],
                             avc, isem)

        def iwait(rowc, avc, isem):
            pltpu.make_async_copy(
                row_hbm.at[pl.ds(0, CH)], rowc, isem).wait()
            pltpu.make_async_copy(
                av_hbm.at[pl.ds(0, CH)], avc, isem).wait()

        pltpu.sync_copy(row_hbm.at[pl.ds(start * CH, CH)], rowca)
        pltpu.sync_copy(av_hbm.at[pl.ds(start * CH, CH)], avca)
        if table_in_spmem:
            for t in range(RPT // CH):
                sl = pl.ds(base_row + t * CH, CH)
                pltpu.sync_copy(y_hbm.at[sl], bufb)
                pltpu.sync_copy(bufb, table.at[sl])
        else:
            gstart(0, bufa, gsema)
        istart(1, rowcb, avcb, isemb)
        plsc.subcore_barrier()
        if table_in_spmem:
            gstart(0, bufa, gsema)

        def half(j, cur_buf, cur_rowc, cur_avc, cur_gsem, cur_isem,
                 oth_buf, oth_rowc, oth_avc, oth_gsem, oth_isem):
            @pl.when(j + 1 < count)
            def _():
                iwait(oth_rowc, oth_avc, oth_isem)
                gstart(j + 1, oth_buf, oth_gsem)
            gwait(cur_buf, cur_gsem)

            def scale(i, c2):
                av = plsc.load_gather(
                    cur_avc, [jnp.full((16,), i, jnp.int32)])
                for t in range(tw):
                    cur_buf[i, pl.ds(t * 16, 16)] = (
                        cur_buf[i, pl.ds(t * 16, 16)] * av)
                return c2

            lax.fori_loop(0, CH, scale, 0, unroll=4)
            pltpu.sync_copy(cur_buf, acc.at[cur_rowc], add=True)

            @pl.when(j + 2 < count)
            def _():
                istart(j + 2, cur_rowc, cur_avc, cur_isem)

        def body(j, carry):
            even = (j % 2) == 0

            @pl.when(even)
            def _():
                half(j, bufa, rowca, avca, gsema, isema,
                     bufb, rowcb, avcb, gsemb, isemb)

            @pl.when(jnp.logical_not(even))
            def _():
                half(j, bufb, rowcb, avcb, gsemb, isemb,
                     bufa, rowca, avca, gsema, isema)

            return carry

        lax.fori_loop(0, count, body, 0)
        plsc.subcore_barrier()
        for t in range(RPT // CH):
            pltpu.sync_copy(acc.at[pl.ds(base_row + t * CH, CH)], bufa)
            pltpu.sync_copy(
                bufa, out_hbm.at[pl.ds(cid * NP + base_row + t * CH, CH)])

    return spmm


_spmm128 = _make_spmm(D, table_in_spmem=False)
_spmm32 = _make_spmm(KP, table_in_spmem=True)


@functools.partial(
    pl.kernel,
    out_type=jax.ShapeDtypeStruct((NW * 16,), jnp.float32),
    mesh=_mesh,
    compiler_params=_sc_params,
    scratch_types=[
        pltpu.VMEM((SLAB * CH,), jnp.int32),
        pltpu.VMEM((SLAB * CH,), jnp.int32),
        pltpu.VMEM((SLAB * CH,), jnp.float32),
        pltpu.VMEM((CH, KP), jnp.float32),
        pltpu.VMEM((CH, KP), jnp.float32),
        pltpu.VMEM((CH, KP), jnp.float32),
        pltpu.VMEM((CH, KP), jnp.float32),
        pltpu.VMEM((16,), jnp.float32),
        pltpu.VMEM_SHARED((NP, KP), jnp.float32),
        pltpu.SemaphoreType.DMA,
        pltpu.SemaphoreType.DMA,
    ],
)
def _ff(fx_hbm, row_hbm, col_hbm, c_hbm, out_hbm,
        rowv, colv, cvv, raa, rba, rab, rbb, pv, fxs, sema, semb):
    cid = lax.axis_index("c")
    sid = lax.axis_index("s")
    wid = sid * NC + cid
    start = wid * CH_BASE + jnp.minimum(wid, CH_EXTRA)
    count = CH_BASE + (wid < CH_EXTRA).astype(jnp.int32)
    lane = lax.iota(jnp.int32, 16)

    pltpu.sync_copy(row_hbm.at[pl.ds(start * CH, SLAB * CH)], rowv)
    pltpu.sync_copy(col_hbm.at[pl.ds(start * CH, SLAB * CH)], colv)
    pltpu.sync_copy(c_hbm.at[pl.ds(start * CH, SLAB * CH)], cvv)
    for t in range(RPT // CH):
        sl = pl.ds(sid * RPT + t * CH, CH)
        pltpu.sync_copy(fx_hbm.at[sl], raa)
        pltpu.sync_copy(raa, fxs.at[sl])
    plsc.subcore_barrier()

    def gstart(j, ra, rb, sem):
        pltpu.async_copy(fxs.at[rowv.at[pl.ds(j * CH, CH)]], ra, sem)
        pltpu.async_copy(fxs.at[colv.at[pl.ds(j * CH, CH)]], rb, sem)

    def gwait(ra, rb, sem):
        pltpu.make_async_copy(fxs.at[rowv.at[pl.ds(0, CH)]], ra, sem).wait()
        pltpu.make_async_copy(fxs.at[colv.at[pl.ds(0, CH)]], rb, sem).wait()

    gstart(0, raa, rba, sema)

    def process(j, ra, rb, sem, part):
        gwait(ra, rb, sem)
        for g in range(CH // 16):
            eidx = lane + (g * 16)
            acc = jnp.zeros((16,), jnp.float32)
            for k in range(K):
                kk = jnp.full((16,), k, jnp.int32)
                acc = acc + (plsc.load_gather(ra, [eidx, kk]) *
                             plsc.load_gather(rb, [eidx, kk]))
            d = acc - cvv[pl.ds(j * CH + g * 16, 16)]
            part = part + d * d
        return part

    def body(j, part):
        even = (j % 2) == 0

        def do_even(p):
            @pl.when(j + 1 < count)
            def _():
                gstart(j + 1, rab, rbb, semb)
            return process(j, raa, rba, sema, p)

        def do_odd(p):
            @pl.when(j + 1 < count)
            def _():
                gstart(j + 1, raa, rba, sema)
            return process(j, rab, rbb, semb, p)

        return lax.cond(even, do_even, do_odd, part)

    part = lax.fori_loop(0, count, body, jnp.zeros((16,), jnp.float32))
    pv[...] = part
    pltpu.sync_copy(pv, out_hbm.at[pl.ds(wid * 16, 16)])


def _tc1_body(x_ref, w1_ref, ws1_ref, b_ref, y_ref, s1_ref):
    x = x_ref[...]
    y_ref[...] = jnp.dot(x, w1_ref[...], preferred_element_type=jnp.float32)
    s1_ref[...] = (jnp.dot(x, ws1_ref[...], preferred_element_type=jnp.float32)
                   + b_ref[...])


def _tc1(X, W1, Ws1, b):
    return pl.pallas_call(
        _tc1_body,
        out_shape=[jax.ShapeDtypeStruct((N, H), jnp.float32),
                   jax.ShapeDtypeStruct((N, H), jnp.float32)],
    )(X, W1, Ws1, b)


def _tc2_body(z2_ref, s1_ref, w2_ref, ws2_ref, bu_ref, g_ref, u_ref):
    z = z2_ref[0, :N, :] + z2_ref[1, :N, :]
    h = jnp.maximum(z + s1_ref[...], 0.0)
    g = jnp.dot(h, w2_ref[...], preferred_element_type=jnp.float32)
    g_ref[...] = jnp.concatenate(
        [g, jnp.zeros((NP - N, KP), jnp.float32)], axis=0)
    u_ref[...] = (jnp.dot(h, ws2_ref[...], preferred_element_type=jnp.float32)
                  + bu_ref[...])


def _tc2(Z2, S1, W2p, Ws2p, bUp):
    return pl.pallas_call(
        _tc2_body,
        out_shape=[jax.ShapeDtypeStruct((NP, KP), jnp.float32),
                   jax.ShapeDtypeStruct((N, KP), jnp.float32)],
    )(Z2, S1, W2p, Ws2p, bUp)


def _tc3_body(v2_ref, u_ref, fx_ref, ssum_ref):
    logits = v2_ref[0, :N, :] + v2_ref[1, :N, :] + u_ref[...]
    colid = lax.broadcasted_iota(jnp.int32, (N, KP), 1)
    mask = colid < K
    lm = jnp.where(mask, logits, -1e30)
    m = jnp.max(lm, axis=1, keepdims=True)
    e = jnp.where(mask, jnp.exp(lm - m), 0.0)
    s = jnp.sum(e, axis=1, keepdims=True)
    fx = e / s
    fx_ref[...] = jnp.concatenate(
        [fx, jnp.zeros((NP - N, KP), jnp.float32)], axis=0)
    nfx = jnp.log(1.0 - fx * fx)
    ssum_ref[...] = jnp.broadcast_to(jnp.sum(nfx, axis=0, keepdims=True),
                                     (8, KP))


def _tc3(V2, U):
    return pl.pallas_call(
        _tc3_body,
        out_shape=[jax.ShapeDtypeStruct((NP, KP), jnp.float32),
                   jax.ShapeDtypeStruct((8, KP), jnp.float32)],
    )(V2, U)


def kernel(X, edge_index, A_vals, C, W1, b1, Ws1, bs1, W2, b2, Ws2, bs2):
    row = edge_index[0]
    col = edge_index[1]
    row1 = jnp.pad(row, (0, CH))
    col1 = jnp.pad(col, (0, CH))
    av1 = jnp.pad(A_vals, (0, CH))
    c1 = jnp.pad(C, (0, CH))
    b1s = (b1 + bs1)[None, :]
    W2p = jnp.pad(W2, ((0, 0), (0, KP - K)))
    Ws2p = jnp.pad(Ws2, ((0, 0), (0, KP - K)))
    bUp = jnp.pad(b2 + bs2, (0, KP - K))[None, :]

    Y, S1 = _tc1(X, W1, Ws1, b1s)
    Z2 = _spmm128(Y, col1, row1, av1)
    G, U = _tc2(Z2.reshape(2, NP, D), S1, W2p, Ws2p, bUp)
    V2 = _spmm32(G, col1, row1, av1)
    FXp, Ssum = _tc3(V2.reshape(2, NP, KP), U)
    ffp = _ff(FXp, row1, col1, c1)

    S = Ssum[0, :K]
    preg = -jnp.sum(jnp.log(1.0001 - jnp.exp(S)))
    loss = jnp.sum(ffp) / E + REG * preg
    return loss


# speedup vs baseline: 1.3066x; 1.3066x over previous
import functools

import jax
import jax.numpy as jnp
from jax import lax
from jax.experimental import pallas as pl
from jax.experimental.pallas import tpu as pltpu
from jax.experimental.pallas import tpu_sc as plsc

N = 10000
E = 320000
D = 128
H = 128
K = 30
KP = 32
REG = 0.01

CH = 128
NCH = E // CH
NC = 2
NS = 16
NW = NC * NS
CH_BASE = NCH // NW
CH_EXTRA = NCH - CH_BASE * NW
SLAB = CH_BASE + 1
NP = 10240
RPT = NP // NS

_mesh = plsc.VectorSubcoreMesh(core_axis_name="c", subcore_axis_name="s")
_sc_params = pltpu.CompilerParams(needs_layout_passes=False,
                                  use_tc_tiling_on_sc=False)


def _make_spmm(width, table_in_spmem):
    tw = width // 16
    scratch = [
        pltpu.VMEM((SLAB * CH,), jnp.int32),
        pltpu.VMEM((CH,), jnp.int32),
        pltpu.VMEM((CH,), jnp.int32),
        pltpu.VMEM((CH,), jnp.float32),
        pltpu.VMEM((CH,), jnp.float32),
        pltpu.VMEM((CH, width), jnp.float32),
        pltpu.VMEM((CH, width), jnp.float32),
        pltpu.VMEM_SHARED((NP, width), jnp.float32),
        pltpu.SemaphoreType.DMA,
        pltpu.SemaphoreType.DMA,
        pltpu.SemaphoreType.DMA,
        pltpu.SemaphoreType.DMA,
    ]
    if table_in_spmem:
        scratch.append(pltpu.VMEM_SHARED((NP, width), jnp.float32))

    @functools.partial(
        pl.kernel,
        out_type=jax.ShapeDtypeStruct((2 * NP, width), jnp.float32),
        mesh=_mesh,
        compiler_params=_sc_params,
        scratch_types=scratch,
    )
    def spmm(y_hbm, col_hbm, row_hbm, av_hbm, out_hbm,
             colv, rowca, rowcb, avca, avcb, bufa, bufb, acc,
             gsema, gsemb, isema, isemb, *rest):
        table = rest[0] if table_in_spmem else y_hbm
        cid = lax.axis_index("c")
        sid = lax.axis_index("s")
        wid = sid * NC + cid
        start = wid * CH_BASE + jnp.minimum(wid, CH_EXTRA)
        count = CH_BASE + (wid < CH_EXTRA).astype(jnp.int32)

        pltpu.sync_copy(col_hbm.at[pl.ds(start * CH, SLAB * CH)], colv)

        zeros16 = jnp.zeros((16,), jnp.float32)

        def zrow(i, carry):
            for t in range(tw):
                bufa[i, pl.ds(t * 16, 16)] = zeros16
            return carry

        lax.fori_loop(0, CH, zrow, 0)
        base_row = sid * RPT
        for t in range(RPT // CH):
            pltpu.sync_copy(bufa, acc.at[pl.ds(base_row + t * CH, CH)])

        def gstart(j, buf, gsem):
            pltpu.async_copy(table.at[colv.at[pl.ds(j * CH, CH)]], buf, gsem)

        def gwait(buf, gsem):
            pltpu.make_async_copy(
                table.at[colv.at[pl.ds(0, CH)]], buf, gsem).wait()

        def istart(j, rowc, avc, isem):
            pltpu.async_copy(row_hbm.at[pl.ds((start + j) * CH, CH)],
                             rowc, isem)
            pltpu.async_copy(av_hbm.at[pl.ds((start + j) * CH, CH)],
                             avc, isem)

        def iwait(rowc, avc, isem):
            pltpu.make_async_copy(
                row_hbm.at[pl.ds(0, CH)], rowc, isem).wait()
            pltpu.make_async_copy(
                av_hbm.at[pl.ds(0, CH)], avc, isem).wait()

        pltpu.sync_copy(row_hbm.at[pl.ds(start * CH, CH)], rowca)
        pltpu.sync_copy(av_hbm.at[pl.ds(start * CH, CH)], avca)
        if table_in_spmem:
            for t in range(RPT // CH):
                sl = pl.ds(base_row + t * CH, CH)
                pltpu.sync_copy(y_hbm.at[sl], bufb)
                pltpu.sync_copy(bufb, table.at[sl])
        else:
            gstart(0, bufa, gsema)
        istart(1, rowcb, avcb, isemb)
        plsc.subcore_barrier()
        if table_in_spmem:
            gstart(0, bufa, gsema)

        def half(j, cur_buf, cur_rowc, cur_avc, cur_gsem, cur_isem,
                 oth_buf, oth_rowc, oth_avc, oth_gsem, oth_isem):
            @pl.when(j + 1 < count)
            def _():
                iwait(oth_rowc, oth_avc, oth_isem)
                gstart(j + 1, oth_buf, oth_gsem)
            gwait(cur_buf, cur_gsem)

            def scale(i, c2):
                av = plsc.load_gather(
                    cur_avc, [jnp.full((16,), i, jnp.int32)])
                for t in range(tw):
                    cur_buf[i, pl.ds(t * 16, 16)] = (
                        cur_buf[i, pl.ds(t * 16, 16)] * av)
                return c2

            lax.fori_loop(0, CH, scale, 0, unroll=4)
            pltpu.sync_copy(cur_buf, acc.at[cur_rowc], add=True)

            @pl.when(j + 2 < count)
            def _():
                istart(j + 2, cur_rowc, cur_avc, cur_isem)

        def body(j, carry):
            even = (j % 2) == 0

            @pl.when(even)
            def _():
                half(j, bufa, rowca, avca, gsema, isema,
                     bufb, rowcb, avcb, gsemb, isemb)

            @pl.when(jnp.logical_not(even))
            def _():
                half(j, bufb, rowcb, avcb, gsemb, isemb,
                     bufa, rowca, avca, gsema, isema)

            return carry

        lax.fori_loop(0, count, body, 0)
        plsc.subcore_barrier()
        for t in range(RPT // CH):
            pltpu.sync_copy(acc.at[pl.ds(base_row + t * CH, CH)], bufa)
            pltpu.sync_copy(
                bufa, out_hbm.at[pl.ds(cid * NP + base_row + t * CH, CH)])

    return spmm


_spmm128 = _make_spmm(D, table_in_spmem=False)
_spmm32 = _make_spmm(KP, table_in_spmem=True)


@functools.partial(
    pl.kernel,
    out_type=jax.ShapeDtypeStruct((NW * 16,), jnp.float32),
    mesh=_mesh,
    compiler_params=_sc_params,
    scratch_types=[
        pltpu.VMEM((SLAB * CH,), jnp.int32),
        pltpu.VMEM((SLAB * CH,), jnp.int32),
        pltpu.VMEM((SLAB * CH,), jnp.float32),
        pltpu.VMEM((SLAB * CH,), jnp.float32),
        pltpu.VMEM((NP,), jnp.float32),
        pltpu.VMEM((NP,), jnp.float32),
        pltpu.VMEM((NP,), jnp.float32),
        pltpu.VMEM((NP,), jnp.float32),
        pltpu.VMEM((16,), jnp.float32),
        pltpu.SemaphoreType.DMA,
        pltpu.SemaphoreType.DMA,
    ],
)
def _ff(fxt_hbm, row_hbm, col_hbm, c_hbm, out_hbm,
        rowv, colv, cvv, accv, fca, fcb, fcc, fcd, pv, sema, semb):
    cid = lax.axis_index("c")
    sid = lax.axis_index("s")
    wid = sid * NC + cid
    start = wid * CH_BASE + jnp.minimum(wid, CH_EXTRA)
    count = CH_BASE + (wid < CH_EXTRA).astype(jnp.int32)
    NG = SLAB * CH // 16

    pltpu.sync_copy(row_hbm.at[pl.ds(start * CH, SLAB * CH)], rowv)
    pltpu.sync_copy(col_hbm.at[pl.ds(start * CH, SLAB * CH)], colv)
    pltpu.sync_copy(c_hbm.at[pl.ds(start * CH, SLAB * CH)], cvv)

    zeros16 = jnp.zeros((16,), jnp.float32)

    def zacc(g, carry):
        accv[pl.ds(g * 16, 16)] = zeros16
        return carry

    lax.fori_loop(0, NG, zacc, 0)

    def kstart(k, ca, cb, sem):
        pltpu.async_copy(fxt_hbm.at[k], ca, sem)
        pltpu.async_copy(fxt_hbm.at[k + 1], cb, sem)

    def kwait(ca, cb, sem):
        pltpu.make_async_copy(fxt_hbm.at[0], ca, sem).wait()
        pltpu.make_async_copy(fxt_hbm.at[0], cb, sem).wait()

    kstart(0, fca, fcb, sema)

    for kp in range(K // 2):
        cur = (fca, fcb, sema) if kp % 2 == 0 else (fcc, fcd, semb)
        oth = (fcc, fcd, semb) if kp % 2 == 0 else (fca, fcb, sema)
        kwait(cur[0], cur[1], cur[2])
        if kp + 1 < K // 2:
            kstart(2 * (kp + 1), oth[0], oth[1], oth[2])
        ca, cb = cur[0], cur[1]

        def kbody(g, carry):
            sl = pl.ds(g * 16, 16)
            r16 = rowv[sl]
            c16 = colv[sl]
            acc = accv[sl]
            acc = acc + (plsc.load_gather(ca, [r16]) *
                         plsc.load_gather(ca, [c16]))
            acc = acc + (plsc.load_gather(cb, [r16]) *
                         plsc.load_gather(cb, [c16]))
            accv[sl] = acc
            return carry

        lax.fori_loop(0, NG, kbody, 0, unroll=4)

    def fin(g, part):
        sl = pl.ds(g * 16, 16)
        d = accv[sl] - cvv[sl]
        return part + d * d

    part = lax.fori_loop(0, count * (CH // 16), fin,
                         jnp.zeros((16,), jnp.float32))
    pv[...] = part
    pltpu.sync_copy(pv, out_hbm.at[pl.ds(wid * 16, 16)])


def _tc1_body(x_ref, w1_ref, ws1_ref, b_ref, y_ref, s1_ref):
    x = x_ref[...]
    y_ref[...] = jnp.dot(x, w1_ref[...], preferred_element_type=jnp.float32)
    s1_ref[...] = (jnp.dot(x, ws1_ref[...], preferred_element_type=jnp.float32)
                   + b_ref[...])


def _tc1(X, W1, Ws1, b):
    return pl.pallas_call(
        _tc1_body,
        out_shape=[jax.ShapeDtypeStruct((N, H), jnp.float32),
                   jax.ShapeDtypeStruct((N, H), jnp.float32)],
    )(X, W1, Ws1, b)


def _tc2_body(z2_ref, s1_ref, w2_ref, ws2_ref, bu_ref, g_ref, u_ref):
    z = z2_ref[0, :N, :] + z2_ref[1, :N, :]
    h = jnp.maximum(z + s1_ref[...], 0.0)
    g = jnp.dot(h, w2_ref[...], preferred_element_type=jnp.float32)
    g_ref[...] = jnp.concatenate(
        [g, jnp.zeros((NP - N, KP), jnp.float32)], axis=0)
    u_ref[...] = (jnp.dot(h, ws2_ref[...], preferred_element_type=jnp.float32)
                  + bu_ref[...])


def _tc2(Z2, S1, W2p, Ws2p, bUp):
    return pl.pallas_call(
        _tc2_body,
        out_shape=[jax.ShapeDtypeStruct((NP, KP), jnp.float32),
                   jax.ShapeDtypeStruct((N, KP), jnp.float32)],
    )(Z2, S1, W2p, Ws2p, bUp)


def _tc3_body(v2_ref, u_ref, fx_ref, ssum_ref):
    logits = v2_ref[0, :N, :] + v2_ref[1, :N, :] + u_ref[...]
    colid = lax.broadcasted_iota(jnp.int32, (N, KP), 1)
    mask = colid < K
    lm = jnp.where(mask, logits, -1e30)
    m = jnp.max(lm, axis=1, keepdims=True)
    e = jnp.where(mask, jnp.exp(lm - m), 0.0)
    s = jnp.sum(e, axis=1, keepdims=True)
    fx = e / s
    fxt = jnp.concatenate([fx, jnp.zeros((NP - N, KP), jnp.float32)], axis=0).T
    fx_ref[...] = fxt
    nfx = jnp.log(1.0 - fx * fx)
    ssum_ref[...] = jnp.broadcast_to(jnp.sum(nfx, axis=0, keepdims=True),
                                     (8, KP))


def _tc3(V2, U):
    return pl.pallas_call(
        _tc3_body,
        out_shape=[jax.ShapeDtypeStruct((KP, NP), jnp.float32),
                   jax.ShapeDtypeStruct((8, KP), jnp.float32)],
    )(V2, U)


def kernel(X, edge_index, A_vals, C, W1, b1, Ws1, bs1, W2, b2, Ws2, bs2):
    row = edge_index[0]
    col = edge_index[1]
    row1 = jnp.pad(row, (0, CH))
    col1 = jnp.pad(col, (0, CH))
    av1 = jnp.pad(A_vals, (0, CH))
    c1 = jnp.pad(C, (0, CH))
    b1s = (b1 + bs1)[None, :]
    W2p = jnp.pad(W2, ((0, 0), (0, KP - K)))
    Ws2p = jnp.pad(Ws2, ((0, 0), (0, KP - K)))
    bUp = jnp.pad(b2 + bs2, (0, KP - K))[None, :]

    Y, S1 = _tc1(X, W1, Ws1, b1s)
    Z2 = _spmm128(Y, col1, row1, av1)
    G, U = _tc2(Z2.reshape(2, NP, D), S1, W2p, Ws2p, bUp)
    V2 = _spmm32(G, col1, row1, av1)
    FXp, Ssum = _tc3(V2.reshape(2, NP, KP), U)
    ffp = _ff(FXp, row1, col1, c1)

    S = Ssum[0, :K]
    preg = -jnp.sum(jnp.log(1.0001 - jnp.exp(S)))
    loss = jnp.sum(ffp) / E + REG * preg
    return loss


# speedup vs baseline: 1.4649x; 1.1211x over previous
import functools

import jax
import jax.numpy as jnp
from jax import lax
from jax.experimental import pallas as pl
from jax.experimental.pallas import tpu as pltpu
from jax.experimental.pallas import tpu_sc as plsc

N = 10000
E = 320000
D = 128
H = 128
K = 30
KP = 32
REG = 0.01

CH = 128
NCH = E // CH
NC = 2
NS = 16
NW = NC * NS
CH_BASE = NCH // NW
CH_EXTRA = NCH - CH_BASE * NW
SLAB = CH_BASE + 1
NP = 10240
RPT = NP // NS

_mesh = plsc.VectorSubcoreMesh(core_axis_name="c", subcore_axis_name="s")
_sc_params = pltpu.CompilerParams(needs_layout_passes=False,
                                  use_tc_tiling_on_sc=False)


def _make_spmm(width, table_in_spmem):
    tw = width // 16
    scratch = [
        pltpu.VMEM((SLAB * CH,), jnp.int32),
        pltpu.VMEM((CH,), jnp.int32),
        pltpu.VMEM((CH,), jnp.int32),
        pltpu.VMEM((CH,), jnp.float32),
        pltpu.VMEM((CH,), jnp.float32),
        pltpu.VMEM((CH, width), jnp.float32),
        pltpu.VMEM((CH, width), jnp.float32),
        pltpu.VMEM_SHARED((NP, width), jnp.float32),
        pltpu.SemaphoreType.DMA,
        pltpu.SemaphoreType.DMA,
        pltpu.SemaphoreType.DMA,
        pltpu.SemaphoreType.DMA,
        pltpu.SemaphoreType.DMA,
        pltpu.SemaphoreType.DMA,
    ]
    if table_in_spmem:
        scratch.append(pltpu.VMEM_SHARED((NP, width), jnp.float32))

    @functools.partial(
        pl.kernel,
        out_type=jax.ShapeDtypeStruct((2 * NP, width), jnp.float32),
        mesh=_mesh,
        compiler_params=_sc_params,
        scratch_types=scratch,
    )
    def spmm(y_hbm, col_hbm, row_hbm, av_hbm, out_hbm,
             colv, rowca, rowcb, avca, avcb, bufa, bufb, acc,
             gsema, gsemb, isema, isemb, ssema, ssemb, *rest):
        table = rest[0] if table_in_spmem else y_hbm
        cid = lax.axis_index("c")
        sid = lax.axis_index("s")
        wid = sid * NC + cid
        start = wid * CH_BASE + jnp.minimum(wid, CH_EXTRA)
        count = CH_BASE + (wid < CH_EXTRA).astype(jnp.int32)

        pltpu.sync_copy(col_hbm.at[pl.ds(start * CH, SLAB * CH)], colv)

        zeros16 = jnp.zeros((16,), jnp.float32)

        def zrow(i, carry):
            for t in range(tw):
                bufa[i, pl.ds(t * 16, 16)] = zeros16
            return carry

        lax.fori_loop(0, CH, zrow, 0)
        base_row = sid * RPT
        for t in range(RPT // CH):
            pltpu.sync_copy(bufa, acc.at[pl.ds(base_row + t * CH, CH)])

        def gstart(j, buf, gsem):
            pltpu.async_copy(table.at[colv.at[pl.ds(j * CH, CH)]], buf, gsem)

        def gwait(buf, gsem):
            pltpu.make_async_copy(
                table.at[colv.at[pl.ds(0, CH)]], buf, gsem).wait()

        def istart(j, rowc, avc, isem):
            pltpu.async_copy(row_hbm.at[pl.ds((start + j) * CH, CH)],
                             rowc, isem)
            pltpu.async_copy(av_hbm.at[pl.ds((start + j) * CH, CH)],
                             avc, isem)

        def iwait(rowc, avc, isem):
            pltpu.make_async_copy(
                row_hbm.at[pl.ds(0, CH)], rowc, isem).wait()
            pltpu.make_async_copy(
                av_hbm.at[pl.ds(0, CH)], avc, isem).wait()

        pltpu.sync_copy(row_hbm.at[pl.ds(start * CH, CH)], rowca)
        pltpu.sync_copy(av_hbm.at[pl.ds(start * CH, CH)], avca)
        if table_in_spmem:
            for t in range(RPT // CH):
                sl = pl.ds(base_row + t * CH, CH)
                pltpu.sync_copy(y_hbm.at[sl], bufb)
                pltpu.sync_copy(bufb, table.at[sl])
        else:
            gstart(0, bufa, gsema)
        istart(1, rowcb, avcb, isemb)
        plsc.subcore_barrier()
        if table_in_spmem:
            gstart(0, bufa, gsema)

        def swait(buf, ssem):
            pltpu.make_async_copy(buf, acc.at[rowca], ssem).wait()

        def half(j, cur_buf, cur_rowc, cur_avc, cur_gsem, cur_isem,
                 cur_ssem, oth_buf, oth_rowc, oth_avc, oth_gsem, oth_isem,
                 oth_ssem):
            @pl.when(j + 1 < count)
            def _():
                iwait(oth_rowc, oth_avc, oth_isem)

                @pl.when(j >= 1)
                def _():
                    swait(oth_buf, oth_ssem)
                gstart(j + 1, oth_buf, oth_gsem)
            gwait(cur_buf, cur_gsem)

            def scale(i, c2):
                av = plsc.load_gather(
                    cur_avc, [jnp.full((16,), i, jnp.int32)])
                for t in range(tw):
                    cur_buf[i, pl.ds(t * 16, 16)] = (
                        cur_buf[i, pl.ds(t * 16, 16)] * av)
                return c2

            lax.fori_loop(0, CH, scale, 0, unroll=4)
            pltpu.async_copy(cur_buf, acc.at[cur_rowc], cur_ssem, add=True)

            @pl.when(j + 2 < count)
            def _():
                istart(j + 2, cur_rowc, cur_avc, cur_isem)

        def body(j, carry):
            even = (j % 2) == 0

            @pl.when(even)
            def _():
                half(j, bufa, rowca, avca, gsema, isema, ssema,
                     bufb, rowcb, avcb, gsemb, isemb, ssemb)

            @pl.when(jnp.logical_not(even))
            def _():
                half(j, bufb, rowcb, avcb, gsemb, isemb, ssemb,
                     bufa, rowca, avca, gsema, isema, ssema)

            return carry

        lax.fori_loop(0, count, body, 0)
        swait(bufa, ssema)
        swait(bufb, ssemb)
        plsc.subcore_barrier()
        for t in range(RPT // CH):
            pltpu.sync_copy(acc.at[pl.ds(base_row + t * CH, CH)], bufa)
            pltpu.sync_copy(
                bufa, out_hbm.at[pl.ds(cid * NP + base_row + t * CH, CH)])

    return spmm


_spmm128 = _make_spmm(D, table_in_spmem=False)
_spmm32 = _make_spmm(KP, table_in_spmem=True)


@functools.partial(
    pl.kernel,
    out_type=jax.ShapeDtypeStruct((NW * 16,), jnp.float32),
    mesh=_mesh,
    compiler_params=_sc_params,
    scratch_types=[
        pltpu.VMEM((SLAB * CH,), jnp.int32),
        pltpu.VMEM((SLAB * CH,), jnp.int32),
        pltpu.VMEM((SLAB * CH,), jnp.float32),
        pltpu.VMEM((SLAB * CH,), jnp.float32),
        pltpu.VMEM((NP,), jnp.float32),
        pltpu.VMEM((NP,), jnp.float32),
        pltpu.VMEM((NP,), jnp.float32),
        pltpu.VMEM((NP,), jnp.float32),
        pltpu.VMEM((16,), jnp.float32),
        pltpu.SemaphoreType.DMA,
        pltpu.SemaphoreType.DMA,
    ],
)
def _ff(fxt_hbm, row_hbm, col_hbm, c_hbm, out_hbm,
        rowv, colv, cvv, accv, fca, fcb, fcc, fcd, pv, sema, semb):
    cid = lax.axis_index("c")
    sid = lax.axis_index("s")
    wid = sid * NC + cid
    start = wid * CH_BASE + jnp.minimum(wid, CH_EXTRA)
    count = CH_BASE + (wid < CH_EXTRA).astype(jnp.int32)
    NG = SLAB * CH // 16

    pltpu.sync_copy(row_hbm.at[pl.ds(start * CH, SLAB * CH)], rowv)
    pltpu.sync_copy(col_hbm.at[pl.ds(start * CH, SLAB * CH)], colv)
    pltpu.sync_copy(c_hbm.at[pl.ds(start * CH, SLAB * CH)], cvv)

    zeros16 = jnp.zeros((16,), jnp.float32)

    def zacc(g, carry):
        accv[pl.ds(g * 16, 16)] = zeros16
        return carry

    lax.fori_loop(0, NG, zacc, 0)

    def kstart(k, ca, cb, sem):
        pltpu.async_copy(fxt_hbm.at[k], ca, sem)
        pltpu.async_copy(fxt_hbm.at[k + 1], cb, sem)

    def kwait(ca, cb, sem):
        pltpu.make_async_copy(fxt_hbm.at[0], ca, sem).wait()
        pltpu.make_async_copy(fxt_hbm.at[0], cb, sem).wait()

    kstart(0, fca, fcb, sema)

    for kp in range(K // 2):
        cur = (fca, fcb, sema) if kp % 2 == 0 else (fcc, fcd, semb)
        oth = (fcc, fcd, semb) if kp % 2 == 0 else (fca, fcb, sema)
        kwait(cur[0], cur[1], cur[2])
        if kp + 1 < K // 2:
            kstart(2 * (kp + 1), oth[0], oth[1], oth[2])
        ca, cb = cur[0], cur[1]

        def kbody(g, carry):
            sl = pl.ds(g * 16, 16)
            r16 = rowv[sl]
            c16 = colv[sl]
            acc = accv[sl]
            acc = acc + (plsc.load_gather(ca, [r16]) *
                         plsc.load_gather(ca, [c16]))
            acc = acc + (plsc.load_gather(cb, [r16]) *
                         plsc.load_gather(cb, [c16]))
            accv[sl] = acc
            return carry

        lax.fori_loop(0, NG, kbody, 0, unroll=8)

    def fin(g, part):
        sl = pl.ds(g * 16, 16)
        d = accv[sl] - cvv[sl]
        return part + d * d

    part = lax.fori_loop(0, count * (CH // 16), fin,
                         jnp.zeros((16,), jnp.float32))
    pv[...] = part
    pltpu.sync_copy(pv, out_hbm.at[pl.ds(wid * 16, 16)])


def _tc2_body(z2_ref, x_ref, w1_ref, ws1_ref, b_ref, w2_ref, ws2_ref,
              bu_ref, g_ref, u_ref):
    z = z2_ref[0, :N, :] + z2_ref[1, :N, :]
    h = jnp.maximum(
        jnp.dot(z, w1_ref[...], preferred_element_type=jnp.float32)
        + jnp.dot(x_ref[...], ws1_ref[...],
                  preferred_element_type=jnp.float32)
        + b_ref[...], 0.0)
    g = jnp.dot(h, w2_ref[...], preferred_element_type=jnp.float32)
    g_ref[...] = jnp.concatenate(
        [g, jnp.zeros((NP - N, KP), jnp.float32)], axis=0)
    u_ref[...] = (jnp.dot(h, ws2_ref[...], preferred_element_type=jnp.float32)
                  + bu_ref[...])


def _tc2(Z2, X, W1, Ws1, b, W2p, Ws2p, bUp):
    return pl.pallas_call(
        _tc2_body,
        out_shape=[jax.ShapeDtypeStruct((NP, KP), jnp.float32),
                   jax.ShapeDtypeStruct((N, KP), jnp.float32)],
    )(Z2, X, W1, Ws1, b, W2p, Ws2p, bUp)


def _tc3_body(v2_ref, u_ref, fx_ref, ssum_ref):
    logits = v2_ref[0, :N, :] + v2_ref[1, :N, :] + u_ref[...]
    colid = lax.broadcasted_iota(jnp.int32, (N, KP), 1)
    mask = colid < K
    lm = jnp.where(mask, logits, -1e30)
    m = jnp.max(lm, axis=1, keepdims=True)
    e = jnp.where(mask, jnp.exp(lm - m), 0.0)
    s = jnp.sum(e, axis=1, keepdims=True)
    fx = e / s
    fxt = jnp.concatenate([fx, jnp.zeros((NP - N, KP), jnp.float32)], axis=0).T
    fx_ref[...] = fxt
    nfx = jnp.log(1.0 - fx * fx)
    ssum_ref[...] = jnp.broadcast_to(jnp.sum(nfx, axis=0, keepdims=True),
                                     (8, KP))


def _tc3(V2, U):
    return pl.pallas_call(
        _tc3_body,
        out_shape=[jax.ShapeDtypeStruct((KP, NP), jnp.float32),
                   jax.ShapeDtypeStruct((8, KP), jnp.float32)],
    )(V2, U)


def kernel(X, edge_index, A_vals, C, W1, b1, Ws1, bs1, W2, b2, Ws2, bs2):
    row = edge_index[0]
    col = edge_index[1]
    row1 = jnp.pad(row, (0, CH))
    col1 = jnp.pad(col, (0, CH))
    av1 = jnp.pad(A_vals, (0, CH))
    c1 = jnp.pad(C, (0, CH))
    b1s = (b1 + bs1)[None, :]
    W2p = jnp.pad(W2, ((0, 0), (0, KP - K)))
    Ws2p = jnp.pad(Ws2, ((0, 0), (0, KP - K)))
    bUp = jnp.pad(b2 + bs2, (0, KP - K))[None, :]

    Z2 = _spmm128(X, col1, row1, av1)
    G, U = _tc2(Z2.reshape(2, NP, D), X, W1, Ws1, b1s, W2p, Ws2p, bUp)
    V2 = _spmm32(G, col1, row1, av1)
    FXp, Ssum = _tc3(V2.reshape(2, NP, KP), U)
    ffp = _ff(FXp, row1, col1, c1)

    S = Ssum[0, :K]
    preg = -jnp.sum(jnp.log(1.0001 - jnp.exp(S)))
    loss = jnp.sum(ffp) / E + REG * preg
    return loss
